# Initial kernel scaffold; baseline (speedup 1.0000x reference)
import jax
import jax.numpy as jnp
from jax.experimental import pallas as pl


def kernel(x, r, h_angle, edge_index, lg_edge_index, Wa, ba, gna, bna, We1, be1, gne1, bne1, We2, be2, gne2, bne2, Wz1, bz1, gnz1, bnz1, Wz2, bz2, gnz2, bnz2, Wg, bg, gng, bng, Wfc, bfc):
    raise NotImplementedError("write your pallas kernel here")


# speedup vs baseline: 1.4652x; 1.4652x over previous
import functools

import jax
import jax.numpy as jnp
from jax.experimental import pallas as pl
from jax.experimental.pallas import tpu as pltpu

_H = 64
_BN_EPS = 1e-5
_SEG_EPS = 1e-6
_F32 = jnp.float32


def _pick_bn(n, cap=4096):
    best = 8
    for c in range(8, cap + 1, 8):
        if n % c == 0:
            best = c
    return best


def _finalize_stats(s, n):
    tot = jnp.sum(s, axis=0)
    mu = tot[0] / n
    var = tot[1] / n - mu * mu
    return jnp.stack([mu, var])


def _stats_of(u):
    return jnp.concatenate(
        [jnp.sum(u, axis=0, keepdims=True), jnp.sum(u * u, axis=0, keepdims=True)],
        axis=0)[None]


def _rbf_mm_body(d_ref, w_ref, b_ref, u_ref, s_ref, *, vmin, spacing, bins):
    c = vmin + jax.lax.broadcasted_iota(jnp.int32, (1, bins), 1).astype(_F32) * spacing
    d = d_ref[...]
    rb = jnp.exp((-1.0 / spacing) * (d - c) ** 2)
    u = jnp.dot(rb, w_ref[...], preferred_element_type=_F32) + b_ref[...]
    u_ref[...] = u
    s_ref[...] = _stats_of(u)


def _rbf_mm(d, w, b, vmin, vmax, bins):
    n = d.shape[0]
    f = w.shape[1]
    bn = _pick_bn(n)
    g = n // bn
    spacing = (vmax - vmin) / (bins - 1)
    body = functools.partial(_rbf_mm_body, vmin=vmin, spacing=spacing, bins=bins)
    u, s = pl.pallas_call(
        body,
        grid=(g,),
        in_specs=[pl.BlockSpec((bn, 1), lambda i: (i, 0)),
                  pl.BlockSpec((bins, f), lambda i: (0, 0)),
                  pl.BlockSpec((1, f), lambda i: (0, 0))],
        out_specs=[pl.BlockSpec((bn, f), lambda i: (i, 0)),
                   pl.BlockSpec((1, 2, f), lambda i: (i, 0, 0))],
        out_shape=[jax.ShapeDtypeStruct((n, f), _F32),
                   jax.ShapeDtypeStruct((g, 2, f), _F32)],
    )(d.reshape(n, 1), w, b.reshape(1, f))
    return u, _finalize_stats(s, n)


def _mm_body(x_ref, w_ref, b_ref, u_ref, s_ref):
    u = jnp.dot(x_ref[...], w_ref[...], preferred_element_type=_F32) + b_ref[...]
    u_ref[...] = u
    s_ref[...] = _stats_of(u)


def _mm_stats(x, w, b):
    n, k = x.shape
    f = w.shape[1]
    bn = _pick_bn(n)
    g = n // bn
    u, s = pl.pallas_call(
        _mm_body,
        grid=(g,),
        in_specs=[pl.BlockSpec((bn, k), lambda i: (i, 0)),
                  pl.BlockSpec((k, f), lambda i: (0, 0)),
                  pl.BlockSpec((1, f), lambda i: (0, 0))],
        out_specs=[pl.BlockSpec((bn, f), lambda i: (i, 0)),
                   pl.BlockSpec((1, 2, f), lambda i: (i, 0, 0))],
        out_shape=[jax.ShapeDtypeStruct((n, f), _F32),
                   jax.ShapeDtypeStruct((g, 2, f), _F32)],
    )(x, w, b.reshape(1, f))
    return u, _finalize_stats(s, n)


def _norm_silu(u, mv):
    t = (u - mv[0:1]) * jax.lax.rsqrt(mv[1:2] + _BN_EPS)
    return t


def _nsm_body(u_ref, mv_ref, g_ref, be_ref, w_ref, b_ref, o_ref, s_ref):
    mu = mv_ref[0:1, :]
    var = mv_ref[1:2, :]
    t = (u_ref[...] - mu) * jax.lax.rsqrt(var + _BN_EPS) * g_ref[...] + be_ref[...]
    t = t * jax.nn.sigmoid(t)
    o = jnp.dot(t, w_ref[...], preferred_element_type=_F32) + b_ref[...]
    o_ref[...] = o
    s_ref[...] = _stats_of(o)


def _norm_silu_mm_stats(u, mv, gain, beta, w, b):
    n, k = u.shape
    f = w.shape[1]
    bn = _pick_bn(n)
    g = n // bn
    o, s = pl.pallas_call(
        _nsm_body,
        grid=(g,),
        in_specs=[pl.BlockSpec((bn, k), lambda i: (i, 0)),
                  pl.BlockSpec((2, k), lambda i: (0, 0)),
                  pl.BlockSpec((1, k), lambda i: (0, 0)),
                  pl.BlockSpec((1, k), lambda i: (0, 0)),
                  pl.BlockSpec((k, f), lambda i: (0, 0)),
                  pl.BlockSpec((1, f), lambda i: (0, 0))],
        out_specs=[pl.BlockSpec((bn, f), lambda i: (i, 0)),
                   pl.BlockSpec((1, 2, f), lambda i: (i, 0, 0))],
        out_shape=[jax.ShapeDtypeStruct((n, f), _F32),
                   jax.ShapeDtypeStruct((g, 2, f), _F32)],
    )(u, mv, gain.reshape(1, k), beta.reshape(1, k), w, b.reshape(1, f))
    return o, _finalize_stats(s, n)


def _ns_body(u_ref, mv_ref, g_ref, be_ref, o_ref):
    mu = mv_ref[0:1, :]
    var = mv_ref[1:2, :]
    t = (u_ref[...] - mu) * jax.lax.rsqrt(var + _BN_EPS) * g_ref[...] + be_ref[...]
    o_ref[...] = t * jax.nn.sigmoid(t)


def _norm_silu_only(u, mv, gain, beta):
    n, k = u.shape
    bn = _pick_bn(n)
    g = n // bn
    return pl.pallas_call(
        _ns_body,
        grid=(g,),
        in_specs=[pl.BlockSpec((bn, k), lambda i: (i, 0)),
                  pl.BlockSpec((2, k), lambda i: (0, 0)),
                  pl.BlockSpec((1, k), lambda i: (0, 0)),
                  pl.BlockSpec((1, k), lambda i: (0, 0))],
        out_specs=pl.BlockSpec((bn, k), lambda i: (i, 0)),
        out_shape=jax.ShapeDtypeStruct((n, k), _F32),
    )(u, mv, gain.reshape(1, k), beta.reshape(1, k))


def _proj_body(x_ref, wa_ref, ba_ref, wb_ref, bb_ref, pa_ref, pb_ref):
    x = x_ref[...]
    pa_ref[...] = jnp.dot(x, wa_ref[...], preferred_element_type=_F32) + ba_ref[...]
    pb_ref[...] = jnp.dot(x, wb_ref[...], preferred_element_type=_F32) + bb_ref[...]


def _proj(x, wa, ba, wb, bb):
    n, k = x.shape
    fa = wa.shape[1]
    fb = wb.shape[1]
    bn = _pick_bn(n)
    g = n // bn
    return pl.pallas_call(
        _proj_body,
        grid=(g,),
        in_specs=[pl.BlockSpec((bn, k), lambda i: (i, 0)),
                  pl.BlockSpec((k, fa), lambda i: (0, 0)),
                  pl.BlockSpec((1, fa), lambda i: (0, 0)),
                  pl.BlockSpec((k, fb), lambda i: (0, 0)),
                  pl.BlockSpec((1, fb), lambda i: (0, 0))],
        out_specs=[pl.BlockSpec((bn, fa), lambda i: (i, 0)),
                   pl.BlockSpec((bn, fb), lambda i: (i, 0))],
        out_shape=[jax.ShapeDtypeStruct((n, fa), _F32),
                   jax.ShapeDtypeStruct((n, fb), _F32)],
    )(x, wa, ba.reshape(1, fa), wb, bb.reshape(1, fb))


def _edge_body(ye_ref, g1_ref, g2_ref, w2_ref, b2_ref, m_ref, sct_ref, s_ref):
    m = (g1_ref[:, :_H] + g2_ref[...]
         + jnp.dot(ye_ref[...], w2_ref[...], preferred_element_type=_F32)
         + b2_ref[...])
    sig = jax.nn.sigmoid(m)
    m_ref[...] = m
    sct_ref[:, :_H] = sig * g1_ref[:, _H:]
    sct_ref[:, _H:] = sig
    s_ref[...] = _stats_of(m)


def _edge_stage(ye, g1, g2, w2, b2):
    e = ye.shape[0]
    bn = _pick_bn(e)
    g = e // bn
    m, sct, s = pl.pallas_call(
        _edge_body,
        grid=(g,),
        in_specs=[pl.BlockSpec((bn, _H), lambda i: (i, 0)),
                  pl.BlockSpec((bn, 2 * _H), lambda i: (i, 0)),
                  pl.BlockSpec((bn, _H), lambda i: (i, 0)),
                  pl.BlockSpec((_H, _H), lambda i: (0, 0)),
                  pl.BlockSpec((1, _H), lambda i: (0, 0))],
        out_specs=[pl.BlockSpec((bn, _H), lambda i: (i, 0)),
                   pl.BlockSpec((bn, 2 * _H), lambda i: (i, 0)),
                   pl.BlockSpec((1, 2, _H), lambda i: (i, 0, 0))],
        out_shape=[jax.ShapeDtypeStruct((e, _H), _F32),
                   jax.ShapeDtypeStruct((e, 2 * _H), _F32),
                   jax.ShapeDtypeStruct((g, 2, _H), _F32)],
    )(ye, g1, g2, w2, b2.reshape(1, _H))
    return m, sct, _finalize_stats(s, e)


def _node1_body(x_ref, seg_ref, w4_ref, b4_ref, v_ref, s_ref):
    h = seg_ref[:, :_H] / (seg_ref[:, _H:] + _SEG_EPS)
    v = jnp.dot(x_ref[...], w4_ref[...], preferred_element_type=_F32) + b4_ref[...] + h
    v_ref[...] = v
    s_ref[...] = _stats_of(v)


def _node1(x, seg, w4, b4):
    n = x.shape[0]
    bn = _pick_bn(n)
    g = n // bn
    v, s = pl.pallas_call(
        _node1_body,
        grid=(g,),
        in_specs=[pl.BlockSpec((bn, _H), lambda i: (i, 0)),
                  pl.BlockSpec((bn, 2 * _H), lambda i: (i, 0)),
                  pl.BlockSpec((_H, _H), lambda i: (0, 0)),
                  pl.BlockSpec((1, _H), lambda i: (0, 0))],
        out_specs=[pl.BlockSpec((bn, _H), lambda i: (i, 0)),
                   pl.BlockSpec((1, 2, _H), lambda i: (i, 0, 0))],
        out_shape=[jax.ShapeDtypeStruct((n, _H), _F32),
                   jax.ShapeDtypeStruct((g, 2, _H), _F32)],
    )(x, seg, w4, b4.reshape(1, _H))
    return v, _finalize_stats(s, n)


def _resid_body(base_ref, v_ref, mv_ref, g_ref, be_ref, o_ref, ps_ref):
    mu = mv_ref[0:1, :]
    var = mv_ref[1:2, :]
    t = (v_ref[...] - mu) * jax.lax.rsqrt(var + _BN_EPS) * g_ref[...] + be_ref[...]
    o = base_ref[...] + t * jax.nn.sigmoid(t)
    o_ref[...] = o
    ps_ref[...] = jnp.sum(o, axis=0, keepdims=True)[None]


def _resid_norm_silu(base, v, mv, gain, beta):
    n, k = base.shape
    bn = _pick_bn(n)
    g = n // bn
    o, ps = pl.pallas_call(
        _resid_body,
        grid=(g,),
        in_specs=[pl.BlockSpec((bn, k), lambda i: (i, 0)),
                  pl.BlockSpec((bn, k), lambda i: (i, 0)),
                  pl.BlockSpec((2, k), lambda i: (0, 0)),
                  pl.BlockSpec((1, k), lambda i: (0, 0)),
                  pl.BlockSpec((1, k), lambda i: (0, 0))],
        out_specs=[pl.BlockSpec((bn, k), lambda i: (i, 0)),
                   pl.BlockSpec((1, 1, k), lambda i: (i, 0, 0))],
        out_shape=[jax.ShapeDtypeStruct((n, k), _F32),
                   jax.ShapeDtypeStruct((g, 1, k), _F32)],
    )(base, v, mv, gain.reshape(1, k), beta.reshape(1, k))
    return o, ps


def _gather_rows(table, idx):
    return jnp.take(table, idx, axis=0)


def _segment_sum_rows(data, idx, n_seg):
    return jax.ops.segment_sum(data, idx, num_segments=n_seg)


def _eggc(xn, ye, src, dst, n_seg, W, b, g, be):
    wa = jnp.concatenate([W[0], W[3]], axis=1)
    ba2 = jnp.concatenate([b[0], b[3]], axis=0)
    p_src, p_dst = _proj(xn, wa, ba2, W[1], b[1])
    g1 = _gather_rows(p_src, src)
    g2 = _gather_rows(p_dst, dst)
    m, sct, mv_m = _edge_stage(ye, g1, g2, W[2], b[2])
    seg = _segment_sum_rows(sct, dst, n_seg)
    v, mv_v = _node1(xn, seg, W[4], b[4])
    x_new, ps = _resid_norm_silu(xn, v, mv_v, g[0], be[0])
    y_new, _ = _resid_norm_silu(ye, m, mv_m, g[1], be[1])
    return x_new, y_new, ps


def kernel(x, r, h_angle, edge_index, lg_edge_index, Wa, ba, gna, bna,
           We1, be1, gne1, bne1, We2, be2, gne2, bne2,
           Wz1, bz1, gnz1, bnz1, Wz2, bz2, gnz2, bnz2,
           Wg, bg, gng, bng, Wfc, bfc):
    n_nodes = x.shape[0]
    n_edges = r.shape[0]
    src, dst = edge_index[0], edge_index[1]
    lsrc, ldst = lg_edge_index[0], lg_edge_index[1]

    u, mv = _rbf_mm(h_angle, Wz1, bz1, -1.0, 1.0, Wz1.shape[0])
    u2, mv2 = _norm_silu_mm_stats(u, mv, gnz1, bnz1, Wz2, bz2)
    z = _norm_silu_only(u2, mv2, gnz2, bnz2)

    ua, mva = _mm_stats(x, Wa, ba)
    xh = _norm_silu_only(ua, mva, gna, bna)

    ub, mvb = _rbf_mm(r, We1, be1, 0.0, 8.0, We1.shape[0])
    ub2, mvb2 = _norm_silu_mm_stats(ub, mvb, gne1, bne1, We2, be2)
    y = _norm_silu_only(ub2, mvb2, gne2, bne2)

    n_alignn = 2
    for i in range(n_alignn):
        xh, m, _ = _eggc(xh, y, src, dst, n_nodes,
                         Wg[2 * i], bg[2 * i], gng[2 * i], bng[2 * i])
        y, z, _ = _eggc(m, z, lsrc, ldst, n_edges,
                        Wg[2 * i + 1], bg[2 * i + 1], gng[2 * i + 1], bng[2 * i + 1])

    ps = None
    n_gcn = 2
    for j in range(n_gcn):
        k = 2 * n_alignn + j
        xh, y, ps = _eggc(xh, y, src, dst, n_nodes,
                          Wg[k], bg[k], gng[k], bng[k])

    hpool = jnp.sum(ps, axis=(0, 1)) / n_nodes
    out = hpool @ Wfc + bfc
    return jnp.squeeze(out)


# speedup vs baseline: 1.7829x; 1.2168x over previous
import functools

import jax
import jax.numpy as jnp
from jax import lax
from jax.experimental import pallas as pl
from jax.experimental.pallas import tpu as pltpu
from jax.experimental.pallas import tpu_sc as plsc

_NC = 2
_NS = 16
_CHUNK = 128

_H = 64
_BN_EPS = 1e-5
_SEG_EPS = 1e-6
_F32 = jnp.float32


def _pick_bn(n, cap=4096):
    best = 8
    for c in range(8, cap + 1, 8):
        if n % c == 0:
            best = c
    return best


def _finalize_stats(s, n):
    tot = jnp.sum(s, axis=0)
    mu = tot[0] / n
    var = tot[1] / n - mu * mu
    return jnp.stack([mu, var])


def _stats_of(u):
    return jnp.concatenate(
        [jnp.sum(u, axis=0, keepdims=True), jnp.sum(u * u, axis=0, keepdims=True)],
        axis=0)[None]


def _rbf_mm_body(d_ref, w_ref, b_ref, u_ref, s_ref, *, vmin, spacing, bins):
    c = vmin + jax.lax.broadcasted_iota(jnp.int32, (1, bins), 1).astype(_F32) * spacing
    d = d_ref[...]
    rb = jnp.exp((-1.0 / spacing) * (d - c) ** 2)
    u = jnp.dot(rb, w_ref[...], preferred_element_type=_F32) + b_ref[...]
    u_ref[...] = u
    s_ref[...] = _stats_of(u)


def _rbf_mm(d, w, b, vmin, vmax, bins):
    n = d.shape[0]
    f = w.shape[1]
    bn = _pick_bn(n)
    g = n // bn
    spacing = (vmax - vmin) / (bins - 1)
    body = functools.partial(_rbf_mm_body, vmin=vmin, spacing=spacing, bins=bins)
    u, s = pl.pallas_call(
        body,
        grid=(g,),
        in_specs=[pl.BlockSpec((bn, 1), lambda i: (i, 0)),
                  pl.BlockSpec((bins, f), lambda i: (0, 0)),
                  pl.BlockSpec((1, f), lambda i: (0, 0))],
        out_specs=[pl.BlockSpec((bn, f), lambda i: (i, 0)),
                   pl.BlockSpec((1, 2, f), lambda i: (i, 0, 0))],
        out_shape=[jax.ShapeDtypeStruct((n, f), _F32),
                   jax.ShapeDtypeStruct((g, 2, f), _F32)],
    )(d.reshape(n, 1), w, b.reshape(1, f))
    return u, _finalize_stats(s, n)


def _mm_body(x_ref, w_ref, b_ref, u_ref, s_ref):
    u = jnp.dot(x_ref[...], w_ref[...], preferred_element_type=_F32) + b_ref[...]
    u_ref[...] = u
    s_ref[...] = _stats_of(u)


def _mm_stats(x, w, b):
    n, k = x.shape
    f = w.shape[1]
    bn = _pick_bn(n)
    g = n // bn
    u, s = pl.pallas_call(
        _mm_body,
        grid=(g,),
        in_specs=[pl.BlockSpec((bn, k), lambda i: (i, 0)),
                  pl.BlockSpec((k, f), lambda i: (0, 0)),
                  pl.BlockSpec((1, f), lambda i: (0, 0))],
        out_specs=[pl.BlockSpec((bn, f), lambda i: (i, 0)),
                   pl.BlockSpec((1, 2, f), lambda i: (i, 0, 0))],
        out_shape=[jax.ShapeDtypeStruct((n, f), _F32),
                   jax.ShapeDtypeStruct((g, 2, f), _F32)],
    )(x, w, b.reshape(1, f))
    return u, _finalize_stats(s, n)


def _norm_silu(u, mv):
    t = (u - mv[0:1]) * jax.lax.rsqrt(mv[1:2] + _BN_EPS)
    return t


def _nsm_body(u_ref, mv_ref, g_ref, be_ref, w_ref, b_ref, o_ref, s_ref):
    mu = mv_ref[0:1, :]
    var = mv_ref[1:2, :]
    t = (u_ref[...] - mu) * jax.lax.rsqrt(var + _BN_EPS) * g_ref[...] + be_ref[...]
    t = t * jax.nn.sigmoid(t)
    o = jnp.dot(t, w_ref[...], preferred_element_type=_F32) + b_ref[...]
    o_ref[...] = o
    s_ref[...] = _stats_of(o)


def _norm_silu_mm_stats(u, mv, gain, beta, w, b):
    n, k = u.shape
    f = w.shape[1]
    bn = _pick_bn(n)
    g = n // bn
    o, s = pl.pallas_call(
        _nsm_body,
        grid=(g,),
        in_specs=[pl.BlockSpec((bn, k), lambda i: (i, 0)),
                  pl.BlockSpec((2, k), lambda i: (0, 0)),
                  pl.BlockSpec((1, k), lambda i: (0, 0)),
                  pl.BlockSpec((1, k), lambda i: (0, 0)),
                  pl.BlockSpec((k, f), lambda i: (0, 0)),
                  pl.BlockSpec((1, f), lambda i: (0, 0))],
        out_specs=[pl.BlockSpec((bn, f), lambda i: (i, 0)),
                   pl.BlockSpec((1, 2, f), lambda i: (i, 0, 0))],
        out_shape=[jax.ShapeDtypeStruct((n, f), _F32),
                   jax.ShapeDtypeStruct((g, 2, f), _F32)],
    )(u, mv, gain.reshape(1, k), beta.reshape(1, k), w, b.reshape(1, f))
    return o, _finalize_stats(s, n)


def _ns_body(u_ref, mv_ref, g_ref, be_ref, o_ref):
    mu = mv_ref[0:1, :]
    var = mv_ref[1:2, :]
    t = (u_ref[...] - mu) * jax.lax.rsqrt(var + _BN_EPS) * g_ref[...] + be_ref[...]
    o_ref[...] = t * jax.nn.sigmoid(t)


def _norm_silu_only(u, mv, gain, beta):
    n, k = u.shape
    bn = _pick_bn(n)
    g = n // bn
    return pl.pallas_call(
        _ns_body,
        grid=(g,),
        in_specs=[pl.BlockSpec((bn, k), lambda i: (i, 0)),
                  pl.BlockSpec((2, k), lambda i: (0, 0)),
                  pl.BlockSpec((1, k), lambda i: (0, 0)),
                  pl.BlockSpec((1, k), lambda i: (0, 0))],
        out_specs=pl.BlockSpec((bn, k), lambda i: (i, 0)),
        out_shape=jax.ShapeDtypeStruct((n, k), _F32),
    )(u, mv, gain.reshape(1, k), beta.reshape(1, k))


def _proj_body(x_ref, wa_ref, ba_ref, wb_ref, bb_ref, pa_ref, pb_ref):
    x = x_ref[...]
    pa_ref[...] = jnp.dot(x, wa_ref[...], preferred_element_type=_F32) + ba_ref[...]
    pb_ref[...] = jnp.dot(x, wb_ref[...], preferred_element_type=_F32) + bb_ref[...]


def _proj(x, wa, ba, wb, bb):
    n, k = x.shape
    fa = wa.shape[1]
    fb = wb.shape[1]
    bn = _pick_bn(n)
    g = n // bn
    return pl.pallas_call(
        _proj_body,
        grid=(g,),
        in_specs=[pl.BlockSpec((bn, k), lambda i: (i, 0)),
                  pl.BlockSpec((k, fa), lambda i: (0, 0)),
                  pl.BlockSpec((1, fa), lambda i: (0, 0)),
                  pl.BlockSpec((k, fb), lambda i: (0, 0)),
                  pl.BlockSpec((1, fb), lambda i: (0, 0))],
        out_specs=[pl.BlockSpec((bn, fa), lambda i: (i, 0)),
                   pl.BlockSpec((bn, fb), lambda i: (i, 0))],
        out_shape=[jax.ShapeDtypeStruct((n, fa), _F32),
                   jax.ShapeDtypeStruct((n, fb), _F32)],
    )(x, wa, ba.reshape(1, fa), wb, bb.reshape(1, fb))


def _edge_body(ye_ref, g1_ref, g2_ref, w2_ref, b2_ref, m_ref, sct_ref, s_ref):
    m = (g1_ref[:, :_H] + g2_ref[:, :_H]
         + jnp.dot(ye_ref[...], w2_ref[...], preferred_element_type=_F32)
         + b2_ref[...])
    sig = jax.nn.sigmoid(m)
    m_ref[...] = m
    sct_ref[:, :_H] = sig * g1_ref[:, _H:]
    sct_ref[:, _H:] = sig
    s_ref[...] = _stats_of(m)


def _edge_stage(ye, g1, g2, w2, b2):
    e = ye.shape[0]
    bn = _pick_bn(e)
    g = e // bn
    m, sct, s = pl.pallas_call(
        _edge_body,
        grid=(g,),
        in_specs=[pl.BlockSpec((bn, _H), lambda i: (i, 0)),
                  pl.BlockSpec((bn, 2 * _H), lambda i: (i, 0)),
                  pl.BlockSpec((bn, 2 * _H), lambda i: (i, 0)),
                  pl.BlockSpec((_H, _H), lambda i: (0, 0)),
                  pl.BlockSpec((1, _H), lambda i: (0, 0))],
        out_specs=[pl.BlockSpec((bn, _H), lambda i: (i, 0)),
                   pl.BlockSpec((bn, 2 * _H), lambda i: (i, 0)),
                   pl.BlockSpec((1, 2, _H), lambda i: (i, 0, 0))],
        out_shape=[jax.ShapeDtypeStruct((e, _H), _F32),
                   jax.ShapeDtypeStruct((e, 2 * _H), _F32),
                   jax.ShapeDtypeStruct((g, 2, _H), _F32)],
    )(ye, g1, g2, w2, b2.reshape(1, _H))
    return m, sct, _finalize_stats(s, e)


def _node1_body(x_ref, seg_ref, w4_ref, b4_ref, v_ref, s_ref):
    h = seg_ref[:, :_H] / (seg_ref[:, _H:] + _SEG_EPS)
    v = jnp.dot(x_ref[...], w4_ref[...], preferred_element_type=_F32) + b4_ref[...] + h
    v_ref[...] = v
    s_ref[...] = _stats_of(v)


def _node1(x, seg, w4, b4):
    n = x.shape[0]
    bn = _pick_bn(n)
    g = n // bn
    v, s = pl.pallas_call(
        _node1_body,
        grid=(g,),
        in_specs=[pl.BlockSpec((bn, _H), lambda i: (i, 0)),
                  pl.BlockSpec((bn, 2 * _H), lambda i: (i, 0)),
                  pl.BlockSpec((_H, _H), lambda i: (0, 0)),
                  pl.BlockSpec((1, _H), lambda i: (0, 0))],
        out_specs=[pl.BlockSpec((bn, _H), lambda i: (i, 0)),
                   pl.BlockSpec((1, 2, _H), lambda i: (i, 0, 0))],
        out_shape=[jax.ShapeDtypeStruct((n, _H), _F32),
                   jax.ShapeDtypeStruct((g, 2, _H), _F32)],
    )(x, seg, w4, b4.reshape(1, _H))
    return v, _finalize_stats(s, n)


def _resid_body(base_ref, v_ref, mv_ref, g_ref, be_ref, o_ref, ps_ref):
    mu = mv_ref[0:1, :]
    var = mv_ref[1:2, :]
    t = (v_ref[...] - mu) * jax.lax.rsqrt(var + _BN_EPS) * g_ref[...] + be_ref[...]
    o = base_ref[...] + t * jax.nn.sigmoid(t)
    o_ref[...] = o
    ps_ref[...] = jnp.sum(o, axis=0, keepdims=True)[None]


def _resid_norm_silu(base, v, mv, gain, beta):
    n, k = base.shape
    bn = _pick_bn(n)
    g = n // bn
    o, ps = pl.pallas_call(
        _resid_body,
        grid=(g,),
        in_specs=[pl.BlockSpec((bn, k), lambda i: (i, 0)),
                  pl.BlockSpec((bn, k), lambda i: (i, 0)),
                  pl.BlockSpec((2, k), lambda i: (0, 0)),
                  pl.BlockSpec((1, k), lambda i: (0, 0)),
                  pl.BlockSpec((1, k), lambda i: (0, 0))],
        out_specs=[pl.BlockSpec((bn, k), lambda i: (i, 0)),
                   pl.BlockSpec((1, 1, k), lambda i: (i, 0, 0))],
        out_shape=[jax.ShapeDtypeStruct((n, k), _F32),
                   jax.ShapeDtypeStruct((g, 1, k), _F32)],
    )(base, v, mv, gain.reshape(1, k), beta.reshape(1, k))
    return o, ps


def _sc_gather_pair(t1, t2, src2, dst2, e):
    d1 = t1.shape[1]
    d2 = t2.shape[1]
    cr = e // _CHUNK
    nw = _NC * _NS
    mesh = plsc.VectorSubcoreMesh(core_axis_name="c", subcore_axis_name="s")

    @functools.partial(
        pl.kernel,
        out_type=[jax.ShapeDtypeStruct((e, d1), _F32),
                  jax.ShapeDtypeStruct((e, d2), _F32)],
        mesh=mesh,
        scratch_types=[pltpu.VMEM((1, _CHUNK), jnp.int32),
                       pltpu.VMEM((1, _CHUNK), jnp.int32),
                       pltpu.VMEM((_CHUNK, d1), _F32),
                       pltpu.VMEM((_CHUNK, d2), _F32),
                       pltpu.SemaphoreType.DMA,
                       pltpu.SemaphoreType.DMA],
    )
    def k(t1_h, t2_h, s_h, d_h, o1_h, o2_h, si, di, r1, r2, sem1, sem2):
        wid = lax.axis_index("s") * _NC + lax.axis_index("c")
        q = cr // nw
        rem = cr % nw
        cnt = jnp.where(wid < rem, q + 1, q)
        start = wid * q + jnp.minimum(wid, rem)

        def body(i, carry):
            crow = start + i
            base = crow * _CHUNK
            pltpu.sync_copy(s_h.at[pl.ds(crow, 1)], si)
            pltpu.sync_copy(d_h.at[pl.ds(crow, 1)], di)
            c1 = pltpu.async_copy(t1_h.at[si.at[0]], r1, sem1)
            c2 = pltpu.async_copy(t2_h.at[di.at[0]], r2, sem2)
            c1.wait()
            c2.wait()
            pltpu.sync_copy(r1, o1_h.at[pl.ds(base, _CHUNK)])
            pltpu.sync_copy(r2, o2_h.at[pl.ds(base, _CHUNK)])
            return carry

        lax.fori_loop(0, cnt, body, 0)

    return k(t1, t2, src2, dst2)


def _sc_segment_sum_nodes(data, idx2, n_seg):
    e, dfull = data.shape
    cg = 32
    cr = e // _CHUNK
    rows_pt = n_seg // _NS
    zrows = 25
    mesh = plsc.VectorSubcoreMesh(core_axis_name="c", subcore_axis_name="s")

    @functools.partial(
        pl.kernel,
        out_type=jax.ShapeDtypeStruct((n_seg, dfull), _F32),
        mesh=mesh,
        scratch_types=[pltpu.VMEM_SHARED((n_seg, cg), _F32),
                       pltpu.VMEM((zrows, cg), _F32),
                       pltpu.VMEM((1, _CHUNK), jnp.int32),
                       pltpu.VMEM((_CHUNK, cg), _F32)],
    )
    def k(d_h, i_h, o_h, acc, zb, iv, dv):
        sc = lax.axis_index("c")
        tid = lax.axis_index("s")
        q = cr // _NS
        rem = cr % _NS
        cnt = jnp.where(tid < rem, q + 1, q)
        start = tid * q + jnp.minimum(tid, rem)
        for rr in range(zrows):
            for cc in range(cg // 16):
                zb[rr, cc * 16:(cc + 1) * 16] = jnp.zeros((16,), _F32)
        for p in range(2):
            col0 = (sc * 2 + p) * cg

            def zbody(j, c):
                pltpu.sync_copy(zb, acc.at[pl.ds(tid * rows_pt + j * zrows, zrows)])
                return c

            lax.fori_loop(0, rows_pt // zrows, zbody, 0)
            plsc.subcore_barrier()

            def sbody(i, c):
                crow = start + i
                base = crow * _CHUNK
                pltpu.sync_copy(i_h.at[pl.ds(crow, 1)], iv)
                pltpu.sync_copy(d_h.at[pl.ds(base, _CHUNK), pl.ds(col0, cg)], dv)
                pltpu.sync_copy(dv, acc.at[iv.at[0]], add=True)
                return c

            lax.fori_loop(0, cnt, sbody, 0)
            plsc.subcore_barrier()
            pltpu.sync_copy(acc.at[pl.ds(tid * rows_pt, rows_pt)],
                            o_h.at[pl.ds(tid * rows_pt, rows_pt), pl.ds(col0, cg)])
            plsc.subcore_barrier()

    return k(data, idx2)


def _segment_sum_rows(data, idx, n_seg):
    return jax.ops.segment_sum(data, idx, num_segments=n_seg)


def _eggc(xn, ye, src2, dst2, dst_flat, n_seg, W, b, g, be, node_scatter):
    e = ye.shape[0]
    wa = jnp.concatenate([W[0], W[3]], axis=1)
    ba2 = jnp.concatenate([b[0], b[3]], axis=0)
    wb = jnp.concatenate([W[1], jnp.zeros_like(W[1])], axis=1)
    bb = jnp.concatenate([b[1], jnp.zeros_like(b[1])], axis=0)
    p_src, p_dst = _proj(xn, wa, ba2, wb, bb)
    g1, g2 = _sc_gather_pair(p_src, p_dst, src2, dst2, e)
    m, sct, mv_m = _edge_stage(ye, g1, g2, W[2], b[2])
    if node_scatter:
        seg = _sc_segment_sum_nodes(sct, dst2, n_seg)
    else:
        seg = _segment_sum_rows(sct, dst_flat, n_seg)
    v, mv_v = _node1(xn, seg, W[4], b[4])
    x_new, ps = _resid_norm_silu(xn, v, mv_v, g[0], be[0])
    y_new, _ = _resid_norm_silu(ye, m, mv_m, g[1], be[1])
    return x_new, y_new, ps


def kernel(x, r, h_angle, edge_index, lg_edge_index, Wa, ba, gna, bna,
           We1, be1, gne1, bne1, We2, be2, gne2, bne2,
           Wz1, bz1, gnz1, bnz1, Wz2, bz2, gnz2, bnz2,
           Wg, bg, gng, bng, Wfc, bfc):
    n_nodes = x.shape[0]
    n_edges = r.shape[0]
    src, dst = edge_index[0], edge_index[1]
    lsrc, ldst = lg_edge_index[0], lg_edge_index[1]
    src2 = src.reshape(-1, _CHUNK)
    dst2 = dst.reshape(-1, _CHUNK)
    lsrc2 = lsrc.reshape(-1, _CHUNK)
    ldst2 = ldst.reshape(-1, _CHUNK)

    u, mv = _rbf_mm(h_angle, Wz1, bz1, -1.0, 1.0, Wz1.shape[0])
    u2, mv2 = _norm_silu_mm_stats(u, mv, gnz1, bnz1, Wz2, bz2)
    z = _norm_silu_only(u2, mv2, gnz2, bnz2)

    ua, mva = _mm_stats(x, Wa, ba)
    xh = _norm_silu_only(ua, mva, gna, bna)

    ub, mvb = _rbf_mm(r, We1, be1, 0.0, 8.0, We1.shape[0])
    ub2, mvb2 = _norm_silu_mm_stats(ub, mvb, gne1, bne1, We2, be2)
    y = _norm_silu_only(ub2, mvb2, gne2, bne2)

    n_alignn = 2
    for i in range(n_alignn):
        xh, m, _ = _eggc(xh, y, src2, dst2, dst, n_nodes,
                         Wg[2 * i], bg[2 * i], gng[2 * i], bng[2 * i], False)
        y, z, _ = _eggc(m, z, lsrc2, ldst2, ldst, n_edges,
                        Wg[2 * i + 1], bg[2 * i + 1], gng[2 * i + 1], bng[2 * i + 1],
                        False)

    ps = None
    n_gcn = 2
    for j in range(n_gcn):
        k = 2 * n_alignn + j
        xh, y, ps = _eggc(xh, y, src2, dst2, dst, n_nodes,
                          Wg[k], bg[k], gng[k], bng[k], False)

    hpool = jnp.sum(ps, axis=(0, 1)) / n_nodes
    out = hpool @ Wfc + bfc
    return jnp.squeeze(out)


# speedup vs baseline: 1.8056x; 1.0127x over previous
import functools

import jax
import jax.numpy as jnp
from jax import lax
from jax.experimental import pallas as pl
from jax.experimental.pallas import tpu as pltpu
from jax.experimental.pallas import tpu_sc as plsc

_NC = 2
_NS = 16
_CHUNK = 128

_H = 64
_BN_EPS = 1e-5
_SEG_EPS = 1e-6
_F32 = jnp.float32


def _pick_bn(n, cap=4096):
    best = 8
    for c in range(8, cap + 1, 8):
        if n % c == 0:
            best = c
    return best


def _finalize_stats(s, n):
    tot = jnp.sum(s, axis=0)
    mu = tot[0] / n
    var = tot[1] / n - mu * mu
    return jnp.stack([mu, var])


def _stats_of(u):
    return jnp.concatenate(
        [jnp.sum(u, axis=0, keepdims=True), jnp.sum(u * u, axis=0, keepdims=True)],
        axis=0)[None]


def _rbf_mm_body(d_ref, w_ref, b_ref, u_ref, s_ref, *, vmin, spacing, bins):
    c = vmin + jax.lax.broadcasted_iota(jnp.int32, (1, bins), 1).astype(_F32) * spacing
    d = d_ref[...]
    rb = jnp.exp((-1.0 / spacing) * (d - c) ** 2)
    u = jnp.dot(rb, w_ref[...], preferred_element_type=_F32) + b_ref[...]
    u_ref[...] = u
    s_ref[...] = _stats_of(u)


def _rbf_mm(d, w, b, vmin, vmax, bins):
    n = d.shape[0]
    f = w.shape[1]
    bn = _pick_bn(n)
    g = n // bn
    spacing = (vmax - vmin) / (bins - 1)
    body = functools.partial(_rbf_mm_body, vmin=vmin, spacing=spacing, bins=bins)
    u, s = pl.pallas_call(
        body,
        grid=(g,),
        in_specs=[pl.BlockSpec((bn, 1), lambda i: (i, 0)),
                  pl.BlockSpec((bins, f), lambda i: (0, 0)),
                  pl.BlockSpec((1, f), lambda i: (0, 0))],
        out_specs=[pl.BlockSpec((bn, f), lambda i: (i, 0)),
                   pl.BlockSpec((1, 2, f), lambda i: (i, 0, 0))],
        out_shape=[jax.ShapeDtypeStruct((n, f), _F32),
                   jax.ShapeDtypeStruct((g, 2, f), _F32)],
    )(d.reshape(n, 1), w, b.reshape(1, f))
    return u, _finalize_stats(s, n)


def _mm_body(x_ref, w_ref, b_ref, u_ref, s_ref):
    u = jnp.dot(x_ref[...], w_ref[...], preferred_element_type=_F32) + b_ref[...]
    u_ref[...] = u
    s_ref[...] = _stats_of(u)


def _mm_stats(x, w, b):
    n, k = x.shape
    f = w.shape[1]
    bn = _pick_bn(n)
    g = n // bn
    u, s = pl.pallas_call(
        _mm_body,
        grid=(g,),
        in_specs=[pl.BlockSpec((bn, k), lambda i: (i, 0)),
                  pl.BlockSpec((k, f), lambda i: (0, 0)),
                  pl.BlockSpec((1, f), lambda i: (0, 0))],
        out_specs=[pl.BlockSpec((bn, f), lambda i: (i, 0)),
                   pl.BlockSpec((1, 2, f), lambda i: (i, 0, 0))],
        out_shape=[jax.ShapeDtypeStruct((n, f), _F32),
                   jax.ShapeDtypeStruct((g, 2, f), _F32)],
    )(x, w, b.reshape(1, f))
    return u, _finalize_stats(s, n)


def _norm_silu(u, mv):
    t = (u - mv[0:1]) * jax.lax.rsqrt(mv[1:2] + _BN_EPS)
    return t


def _nsm_body(u_ref, mv_ref, g_ref, be_ref, w_ref, b_ref, o_ref, s_ref):
    mu = mv_ref[0:1, :]
    var = mv_ref[1:2, :]
    t = (u_ref[...] - mu) * jax.lax.rsqrt(var + _BN_EPS) * g_ref[...] + be_ref[...]
    t = t * jax.nn.sigmoid(t)
    o = jnp.dot(t, w_ref[...], preferred_element_type=_F32) + b_ref[...]
    o_ref[...] = o
    s_ref[...] = _stats_of(o)


def _norm_silu_mm_stats(u, mv, gain, beta, w, b):
    n, k = u.shape
    f = w.shape[1]
    bn = _pick_bn(n)
    g = n // bn
    o, s = pl.pallas_call(
        _nsm_body,
        grid=(g,),
        in_specs=[pl.BlockSpec((bn, k), lambda i: (i, 0)),
                  pl.BlockSpec((2, k), lambda i: (0, 0)),
                  pl.BlockSpec((1, k), lambda i: (0, 0)),
                  pl.BlockSpec((1, k), lambda i: (0, 0)),
                  pl.BlockSpec((k, f), lambda i: (0, 0)),
                  pl.BlockSpec((1, f), lambda i: (0, 0))],
        out_specs=[pl.BlockSpec((bn, f), lambda i: (i, 0)),
                   pl.BlockSpec((1, 2, f), lambda i: (i, 0, 0))],
        out_shape=[jax.ShapeDtypeStruct((n, f), _F32),
                   jax.ShapeDtypeStruct((g, 2, f), _F32)],
    )(u, mv, gain.reshape(1, k), beta.reshape(1, k), w, b.reshape(1, f))
    return o, _finalize_stats(s, n)


def _ns_body(u_ref, mv_ref, g_ref, be_ref, o_ref):
    mu = mv_ref[0:1, :]
    var = mv_ref[1:2, :]
    t = (u_ref[...] - mu) * jax.lax.rsqrt(var + _BN_EPS) * g_ref[...] + be_ref[...]
    o_ref[...] = t * jax.nn.sigmoid(t)


def _norm_silu_only(u, mv, gain, beta):
    n, k = u.shape
    bn = _pick_bn(n)
    g = n // bn
    return pl.pallas_call(
        _ns_body,
        grid=(g,),
        in_specs=[pl.BlockSpec((bn, k), lambda i: (i, 0)),
                  pl.BlockSpec((2, k), lambda i: (0, 0)),
                  pl.BlockSpec((1, k), lambda i: (0, 0)),
                  pl.BlockSpec((1, k), lambda i: (0, 0))],
        out_specs=pl.BlockSpec((bn, k), lambda i: (i, 0)),
        out_shape=jax.ShapeDtypeStruct((n, k), _F32),
    )(u, mv, gain.reshape(1, k), beta.reshape(1, k))


def _proj_body(x_ref, wa_ref, ba_ref, wb_ref, bb_ref, pa_ref, pb_ref):
    x = x_ref[...]
    pa_ref[...] = jnp.dot(x, wa_ref[...], preferred_element_type=_F32) + ba_ref[...]
    pb_ref[...] = jnp.dot(x, wb_ref[...], preferred_element_type=_F32) + bb_ref[...]


def _proj(x, wa, ba, wb, bb):
    n, k = x.shape
    fa = wa.shape[1]
    fb = wb.shape[1]
    bn = _pick_bn(n)
    g = n // bn
    return pl.pallas_call(
        _proj_body,
        grid=(g,),
        in_specs=[pl.BlockSpec((bn, k), lambda i: (i, 0)),
                  pl.BlockSpec((k, fa), lambda i: (0, 0)),
                  pl.BlockSpec((1, fa), lambda i: (0, 0)),
                  pl.BlockSpec((k, fb), lambda i: (0, 0)),
                  pl.BlockSpec((1, fb), lambda i: (0, 0))],
        out_specs=[pl.BlockSpec((bn, fa), lambda i: (i, 0)),
                   pl.BlockSpec((bn, fb), lambda i: (i, 0))],
        out_shape=[jax.ShapeDtypeStruct((n, fa), _F32),
                   jax.ShapeDtypeStruct((n, fb), _F32)],
    )(x, wa, ba.reshape(1, fa), wb, bb.reshape(1, fb))


def _edge_body(ye_ref, g1_ref, g2_ref, w2_ref, b2_ref, m_ref, sct_ref, s_ref):
    m = (g1_ref[:, :_H] + g2_ref[:, :_H]
         + jnp.dot(ye_ref[...], w2_ref[...], preferred_element_type=_F32)
         + b2_ref[...])
    sig = jax.nn.sigmoid(m)
    m_ref[...] = m
    sct_ref[:, :_H] = sig * g1_ref[:, _H:]
    sct_ref[:, _H:] = sig
    s_ref[...] = _stats_of(m)


def _edge_stage(ye, g1, g2, w2, b2):
    e = ye.shape[0]
    bn = _pick_bn(e)
    g = e // bn
    m, sct, s = pl.pallas_call(
        _edge_body,
        grid=(g,),
        in_specs=[pl.BlockSpec((bn, _H), lambda i: (i, 0)),
                  pl.BlockSpec((bn, 2 * _H), lambda i: (i, 0)),
                  pl.BlockSpec((bn, 2 * _H), lambda i: (i, 0)),
                  pl.BlockSpec((_H, _H), lambda i: (0, 0)),
                  pl.BlockSpec((1, _H), lambda i: (0, 0))],
        out_specs=[pl.BlockSpec((bn, _H), lambda i: (i, 0)),
                   pl.BlockSpec((bn, 2 * _H), lambda i: (i, 0)),
                   pl.BlockSpec((1, 2, _H), lambda i: (i, 0, 0))],
        out_shape=[jax.ShapeDtypeStruct((e, _H), _F32),
                   jax.ShapeDtypeStruct((e, 2 * _H), _F32),
                   jax.ShapeDtypeStruct((g, 2, _H), _F32)],
    )(ye, g1, g2, w2, b2.reshape(1, _H))
    return m, sct, _finalize_stats(s, e)


def _node1_body(x_ref, seg_ref, w4_ref, b4_ref, v_ref, s_ref):
    h = seg_ref[:, :_H] / (seg_ref[:, _H:] + _SEG_EPS)
    v = jnp.dot(x_ref[...], w4_ref[...], preferred_element_type=_F32) + b4_ref[...] + h
    v_ref[...] = v
    s_ref[...] = _stats_of(v)


def _node1(x, seg, w4, b4):
    n = x.shape[0]
    bn = _pick_bn(n)
    g = n // bn
    v, s = pl.pallas_call(
        _node1_body,
        grid=(g,),
        in_specs=[pl.BlockSpec((bn, _H), lambda i: (i, 0)),
                  pl.BlockSpec((bn, 2 * _H), lambda i: (i, 0)),
                  pl.BlockSpec((_H, _H), lambda i: (0, 0)),
                  pl.BlockSpec((1, _H), lambda i: (0, 0))],
        out_specs=[pl.BlockSpec((bn, _H), lambda i: (i, 0)),
                   pl.BlockSpec((1, 2, _H), lambda i: (i, 0, 0))],
        out_shape=[jax.ShapeDtypeStruct((n, _H), _F32),
                   jax.ShapeDtypeStruct((g, 2, _H), _F32)],
    )(x, seg, w4, b4.reshape(1, _H))
    return v, _finalize_stats(s, n)


def _resid_body(base_ref, v_ref, mv_ref, g_ref, be_ref, o_ref, ps_ref):
    mu = mv_ref[0:1, :]
    var = mv_ref[1:2, :]
    t = (v_ref[...] - mu) * jax.lax.rsqrt(var + _BN_EPS) * g_ref[...] + be_ref[...]
    o = base_ref[...] + t * jax.nn.sigmoid(t)
    o_ref[...] = o
    ps_ref[...] = jnp.sum(o, axis=0, keepdims=True)[None]


def _resid_norm_silu(base, v, mv, gain, beta):
    n, k = base.shape
    bn = _pick_bn(n)
    g = n // bn
    o, ps = pl.pallas_call(
        _resid_body,
        grid=(g,),
        in_specs=[pl.BlockSpec((bn, k), lambda i: (i, 0)),
                  pl.BlockSpec((bn, k), lambda i: (i, 0)),
                  pl.BlockSpec((2, k), lambda i: (0, 0)),
                  pl.BlockSpec((1, k), lambda i: (0, 0)),
                  pl.BlockSpec((1, k), lambda i: (0, 0))],
        out_specs=[pl.BlockSpec((bn, k), lambda i: (i, 0)),
                   pl.BlockSpec((1, 1, k), lambda i: (i, 0, 0))],
        out_shape=[jax.ShapeDtypeStruct((n, k), _F32),
                   jax.ShapeDtypeStruct((g, 1, k), _F32)],
    )(base, v, mv, gain.reshape(1, k), beta.reshape(1, k))
    return o, ps


def _sc_gather_pair(t1, t2, src2, dst2, e):
    d1 = t1.shape[1]
    d2 = t2.shape[1]
    cr = e // _CHUNK
    nw = _NC * _NS
    qs = -(-cr // nw)
    if qs % 2:
        qs += 1
    pairs = qs // 2
    mesh = plsc.VectorSubcoreMesh(core_axis_name="c", subcore_axis_name="s")

    @functools.partial(
        pl.kernel,
        out_type=[jax.ShapeDtypeStruct((e, d1), _F32),
                  jax.ShapeDtypeStruct((e, d2), _F32)],
        mesh=mesh,
        scratch_types=[pltpu.VMEM((1, _CHUNK), jnp.int32),
                       pltpu.VMEM((1, _CHUNK), jnp.int32),
                       pltpu.VMEM((1, _CHUNK), jnp.int32),
                       pltpu.VMEM((1, _CHUNK), jnp.int32),
                       pltpu.VMEM((_CHUNK, d1), _F32),
                       pltpu.VMEM((_CHUNK, d2), _F32),
                       pltpu.VMEM((_CHUNK, d1), _F32),
                       pltpu.VMEM((_CHUNK, d2), _F32),
                       pltpu.SemaphoreType.DMA,
                       pltpu.SemaphoreType.DMA,
                       pltpu.SemaphoreType.DMA,
                       pltpu.SemaphoreType.DMA],
    )
    def k(t1_h, t2_h, s_h, d_h, o1_h, o2_h,
          sia, dia, sib, dib, r1a, r2a, r1b, r2b, gsa, gsb, wsa, wsb):
        wid = lax.axis_index("s") * _NC + lax.axis_index("c")
        start = jnp.minimum(wid * qs, cr - qs)

        def gstart(crow, si, di, r1, r2, gsem):
            pltpu.sync_copy(s_h.at[pl.ds(crow, 1)], si)
            pltpu.sync_copy(d_h.at[pl.ds(crow, 1)], di)
            pltpu.async_copy(t1_h.at[si.at[0]], r1, gsem)
            pltpu.async_copy(t2_h.at[di.at[0]], r2, gsem)

        def gwait(r1, r2, gsem):
            pltpu.make_async_copy(t1_h.at[pl.ds(0, _CHUNK)], r1, gsem).wait()
            pltpu.make_async_copy(t2_h.at[pl.ds(0, _CHUNK)], r2, gsem).wait()

        def wstart(crow, r1, r2, wsem):
            base = crow * _CHUNK
            pltpu.async_copy(r1, o1_h.at[pl.ds(base, _CHUNK)], wsem)
            pltpu.async_copy(r2, o2_h.at[pl.ds(base, _CHUNK)], wsem)

        def wwait(r1, r2, wsem):
            pltpu.make_async_copy(r1, o1_h.at[pl.ds(0, _CHUNK)], wsem).wait()
            pltpu.make_async_copy(r2, o2_h.at[pl.ds(0, _CHUNK)], wsem).wait()

        gstart(start, sia, dia, r1a, r2a, gsa)
        gstart(start + 1, sib, dib, r1b, r2b, gsb)
        gwait(r1a, r2a, gsa)
        wstart(start, r1a, r2a, wsa)
        wwait(r1a, r2a, wsa)
        gstart(start + 2, sia, dia, r1a, r2a, gsa)
        gwait(r1b, r2b, gsb)
        wstart(start + 1, r1b, r2b, wsb)

        def step(t, carry):
            c0 = start + 2 * t
            wwait(r1b, r2b, wsb)
            gstart(c0 + 1, sib, dib, r1b, r2b, gsb)
            gwait(r1a, r2a, gsa)
            wstart(c0, r1a, r2a, wsa)
            wwait(r1a, r2a, wsa)
            nxt = jnp.minimum(c0 + 2, start + qs - 1)
            gstart(nxt, sia, dia, r1a, r2a, gsa)
            gwait(r1b, r2b, gsb)
            wstart(c0 + 1, r1b, r2b, wsb)
            return carry

        lax.fori_loop(1, pairs, step, 0)
        wwait(r1b, r2b, wsb)
        gwait(r1a, r2a, gsa)

    return k(t1, t2, src2, dst2)


def _sc_segment_sum_nodes(data, idx2, n_seg):
    e, dfull = data.shape
    cg = 32
    cr = e // _CHUNK
    rows_pt = n_seg // _NS
    zrows = 25
    mesh = plsc.VectorSubcoreMesh(core_axis_name="c", subcore_axis_name="s")

    @functools.partial(
        pl.kernel,
        out_type=jax.ShapeDtypeStruct((n_seg, dfull), _F32),
        mesh=mesh,
        scratch_types=[pltpu.VMEM_SHARED((n_seg, cg), _F32),
                       pltpu.VMEM((zrows, cg), _F32),
                       pltpu.VMEM((1, _CHUNK), jnp.int32),
                       pltpu.VMEM((_CHUNK, cg), _F32)],
    )
    def k(d_h, i_h, o_h, acc, zb, iv, dv):
        sc = lax.axis_index("c")
        tid = lax.axis_index("s")
        q = cr // _NS
        rem = cr % _NS
        cnt = jnp.where(tid < rem, q + 1, q)
        start = tid * q + jnp.minimum(tid, rem)
        for rr in range(zrows):
            for cc in range(cg // 16):
                zb[rr, cc * 16:(cc + 1) * 16] = jnp.zeros((16,), _F32)
        for p in range(2):
            col0 = (sc * 2 + p) * cg

            def zbody(j, c):
                pltpu.sync_copy(zb, acc.at[pl.ds(tid * rows_pt + j * zrows, zrows)])
                return c

            lax.fori_loop(0, rows_pt // zrows, zbody, 0)
            plsc.subcore_barrier()

            def sbody(i, c):
                crow = start + i
                base = crow * _CHUNK
                pltpu.sync_copy(i_h.at[pl.ds(crow, 1)], iv)
                pltpu.sync_copy(d_h.at[pl.ds(base, _CHUNK), pl.ds(col0, cg)], dv)
                pltpu.sync_copy(dv, acc.at[iv.at[0]], add=True)
                return c

            lax.fori_loop(0, cnt, sbody, 0)
            plsc.subcore_barrier()
            pltpu.sync_copy(acc.at[pl.ds(tid * rows_pt, rows_pt)],
                            o_h.at[pl.ds(tid * rows_pt, rows_pt), pl.ds(col0, cg)])
            plsc.subcore_barrier()

    return k(data, idx2)


def _segment_sum_rows(data, idx, n_seg):
    return jax.ops.segment_sum(data, idx, num_segments=n_seg)


def _eggc(xn, ye, src2, dst2, dst_flat, n_seg, W, b, g, be, node_scatter):
    e = ye.shape[0]
    wa = jnp.concatenate([W[0], W[3]], axis=1)
    ba2 = jnp.concatenate([b[0], b[3]], axis=0)
    wb = jnp.concatenate([W[1], jnp.zeros_like(W[1])], axis=1)
    bb = jnp.concatenate([b[1], jnp.zeros_like(b[1])], axis=0)
    p_src, p_dst = _proj(xn, wa, ba2, wb, bb)
    g1, g2 = _sc_gather_pair(p_src, p_dst, src2, dst2, e)
    m, sct, mv_m = _edge_stage(ye, g1, g2, W[2], b[2])
    if node_scatter:
        seg = _sc_segment_sum_nodes(sct, dst2, n_seg)
    else:
        seg = _segment_sum_rows(sct, dst_flat, n_seg)
    v, mv_v = _node1(xn, seg, W[4], b[4])
    x_new, ps = _resid_norm_silu(xn, v, mv_v, g[0], be[0])
    y_new, _ = _resid_norm_silu(ye, m, mv_m, g[1], be[1])
    return x_new, y_new, ps


def kernel(x, r, h_angle, edge_index, lg_edge_index, Wa, ba, gna, bna,
           We1, be1, gne1, bne1, We2, be2, gne2, bne2,
           Wz1, bz1, gnz1, bnz1, Wz2, bz2, gnz2, bnz2,
           Wg, bg, gng, bng, Wfc, bfc):
    n_nodes = x.shape[0]
    n_edges = r.shape[0]
    src, dst = edge_index[0], edge_index[1]
    lsrc, ldst = lg_edge_index[0], lg_edge_index[1]
    src2 = src.reshape(-1, _CHUNK)
    dst2 = dst.reshape(-1, _CHUNK)
    lsrc2 = lsrc.reshape(-1, _CHUNK)
    ldst2 = ldst.reshape(-1, _CHUNK)

    u, mv = _rbf_mm(h_angle, Wz1, bz1, -1.0, 1.0, Wz1.shape[0])
    u2, mv2 = _norm_silu_mm_stats(u, mv, gnz1, bnz1, Wz2, bz2)
    z = _norm_silu_only(u2, mv2, gnz2, bnz2)

    ua, mva = _mm_stats(x, Wa, ba)
    xh = _norm_silu_only(ua, mva, gna, bna)

    ub, mvb = _rbf_mm(r, We1, be1, 0.0, 8.0, We1.shape[0])
    ub2, mvb2 = _norm_silu_mm_stats(ub, mvb, gne1, bne1, We2, be2)
    y = _norm_silu_only(ub2, mvb2, gne2, bne2)

    n_alignn = 2
    for i in range(n_alignn):
        xh, m, _ = _eggc(xh, y, src2, dst2, dst, n_nodes,
                         Wg[2 * i], bg[2 * i], gng[2 * i], bng[2 * i], False)
        y, z, _ = _eggc(m, z, lsrc2, ldst2, ldst, n_edges,
                        Wg[2 * i + 1], bg[2 * i + 1], gng[2 * i + 1], bng[2 * i + 1],
                        False)

    ps = None
    n_gcn = 2
    for j in range(n_gcn):
        k = 2 * n_alignn + j
        xh, y, ps = _eggc(xh, y, src2, dst2, dst, n_nodes,
                          Wg[k], bg[k], gng[k], bng[k], False)

    hpool = jnp.sum(ps, axis=(0, 1)) / n_nodes
    out = hpool @ Wfc + bfc
    return jnp.squeeze(out)
